# baseline (speedup 1.0000x reference)
import functools

import jax
import jax.numpy as jnp
from jax import lax
from jax.experimental import pallas as pl
from jax.experimental.pallas import tpu as pltpu
from jax.experimental.pallas import tpu_sc as plsc

BATCH = 16384
VOCAB = 1000
L = 16
NC, NS = 2, 16
NW = NC * NS
CHUNK = BATCH // NW


def _lr_body(uid_hbm, iid_hbm, hs_u_hbm, hs_i_hbm, bu_hbm, cu_hbm,
             bi_hbm, ci_hbm, wu_hbm, wi_hbm, out_hbm,
             hs_u_v, hs_i_v, bu_v, cu_v, bi_v, ci_v, wu_v, wi_v,
             uid_v, iid_v, out_v, sem_u, sem_i):
    wid = lax.axis_index("s") * NC + lax.axis_index("c")
    base = wid * CHUNK

    copies_u = [
        pltpu.async_copy(uid_hbm.at[pl.ds(base, CHUNK)], uid_v, sem_u),
        pltpu.async_copy(hs_u_hbm, hs_u_v, sem_u),
        pltpu.async_copy(bu_hbm, bu_v, sem_u),
        pltpu.async_copy(cu_hbm, cu_v, sem_u),
        pltpu.async_copy(wu_hbm, wu_v, sem_u),
    ]
    copies_i = [
        pltpu.async_copy(iid_hbm.at[pl.ds(base, CHUNK)], iid_v, sem_i),
        pltpu.async_copy(hs_i_hbm, hs_i_v, sem_i),
        pltpu.async_copy(bi_hbm, bi_v, sem_i),
        pltpu.async_copy(ci_hbm, ci_v, sem_i),
        pltpu.async_copy(wi_hbm, wi_v, sem_i),
    ]
    for c in copies_u:
        c.wait()

    @plsc.parallel_loop(0, CHUNK, step=L, unroll=4)
    def _(i):
        s = pl.ds(i, L)
        iu = uid_v[s] - 1
        yu = (plsc.load_gather(bu_v, [iu]) * plsc.load_gather(hs_u_v, [iu])
              + plsc.load_gather(cu_v, [iu]))
        out_v[s] = plsc.load_gather(wu_v, [iu]) * yu

    for c in copies_i:
        c.wait()

    @plsc.parallel_loop(0, CHUNK, step=L, unroll=4)
    def _(i):
        s = pl.ds(i, L)
        ii = iid_v[s] - 1
        yi = (plsc.load_gather(bi_v, [ii]) * plsc.load_gather(hs_i_v, [ii])
              + plsc.load_gather(ci_v, [ii]))
        out_v[s] = out_v[s] + plsc.load_gather(wi_v, [ii]) * yi

    pltpu.sync_copy(out_v, out_hbm.at[pl.ds(base, CHUNK)])


@functools.partial(
    pl.kernel,
    out_type=jax.ShapeDtypeStruct((BATCH,), jnp.float32),
    mesh=plsc.VectorSubcoreMesh(core_axis_name="c", subcore_axis_name="s"),
    compiler_params=pltpu.CompilerParams(needs_layout_passes=False),
    scratch_types=[pltpu.VMEM((VOCAB,), jnp.float32) for _ in range(8)]
    + [pltpu.VMEM((CHUNK,), jnp.int32) for _ in range(2)]
    + [pltpu.VMEM((CHUNK,), jnp.float32),
       pltpu.SemaphoreType.DMA, pltpu.SemaphoreType.DMA],
)
def _lr_kernel(*refs):
    _lr_body(*refs)


def kernel(user_id, item_id, user_hs, item_hs, beta_u, bias_u,
           beta_i, bias_i, user_weight, item_weight):
    out = _lr_kernel(user_id, item_id, user_hs.reshape(-1), item_hs.reshape(-1),
                     beta_u.reshape(-1), bias_u.reshape(-1),
                     beta_i.reshape(-1), bias_i.reshape(-1),
                     user_weight.reshape(-1), item_weight.reshape(-1))
    return out.reshape(BATCH, 1)


# speedup vs baseline: 1.0592x; 1.0592x over previous
import functools

import jax
import jax.numpy as jnp
from jax import lax
from jax.experimental import pallas as pl
from jax.experimental.pallas import tpu as pltpu
from jax.experimental.pallas import tpu_sc as plsc

BATCH = 16384
VOCAB = 1000
L = 16
NC, NS = 2, 16
NW = NC * NS
CHUNK = BATCH // NW
VFULL = (VOCAB // L) * L
VTAIL = VOCAB - L


def _lr_body(uid_hbm, iid_hbm, hs_u_hbm, hs_i_hbm, bu_hbm, cu_hbm,
             bi_hbm, ci_hbm, wu_hbm, wi_hbm, out_hbm,
             hs_v, b_v, c_v, w_v, fused_v, tab_u_v, tab_i_v,
             uid_v, iid_v, out_v, shared, sem, sem_ids):
    sid = lax.axis_index("s")
    cid = lax.axis_index("c")
    wid = sid * NC + cid
    base = wid * CHUNK

    id_copies = [
        pltpu.async_copy(uid_hbm.at[pl.ds(base, CHUNK)], uid_v, sem_ids),
        pltpu.async_copy(iid_hbm.at[pl.ds(base, CHUNK)], iid_v, sem_ids),
    ]

    def fuse_tables(hs_hbm, b_hbm, c_hbm, w_hbm, row):
        copies = [
            pltpu.async_copy(hs_hbm, hs_v, sem),
            pltpu.async_copy(b_hbm, b_v, sem),
            pltpu.async_copy(c_hbm, c_v, sem),
            pltpu.async_copy(w_hbm, w_v, sem),
        ]
        for cp in copies:
            cp.wait()

        def fuse(s):
            fused_v[s] = w_v[s] * (b_v[s] * hs_v[s] + c_v[s])

        @plsc.parallel_loop(0, VFULL, step=L, unroll=4)
        def _(j):
            fuse(pl.ds(j, L))

        fuse(pl.ds(VTAIL, L))
        pltpu.sync_copy(fused_v, shared.at[row])

    @pl.when(sid == 0)
    def _():
        fuse_tables(hs_u_hbm, bu_hbm, cu_hbm, wu_hbm, 0)

    @pl.when(sid == 1)
    def _():
        fuse_tables(hs_i_hbm, bi_hbm, ci_hbm, wi_hbm, 1)

    plsc.subcore_barrier()

    tab_copies = [
        pltpu.async_copy(shared.at[0], tab_u_v, sem),
        pltpu.async_copy(shared.at[1], tab_i_v, sem),
    ]
    for cp in tab_copies:
        cp.wait()
    for cp in id_copies:
        cp.wait()

    @plsc.parallel_loop(0, CHUNK, step=L, unroll=4)
    def _(i):
        s = pl.ds(i, L)
        out_v[s] = (plsc.load_gather(tab_u_v, [uid_v[s] - 1])
                    + plsc.load_gather(tab_i_v, [iid_v[s] - 1]))

    pltpu.sync_copy(out_v, out_hbm.at[pl.ds(base, CHUNK)])


@functools.partial(
    pl.kernel,
    out_type=jax.ShapeDtypeStruct((BATCH,), jnp.float32),
    mesh=plsc.VectorSubcoreMesh(core_axis_name="c", subcore_axis_name="s"),
    compiler_params=pltpu.CompilerParams(needs_layout_passes=False),
    scratch_types=[pltpu.VMEM((VOCAB,), jnp.float32) for _ in range(7)]
    + [pltpu.VMEM((CHUNK,), jnp.int32) for _ in range(2)]
    + [pltpu.VMEM((CHUNK,), jnp.float32),
       pltpu.VMEM_SHARED((2, VOCAB), jnp.float32),
       pltpu.SemaphoreType.DMA, pltpu.SemaphoreType.DMA],
)
def _lr_kernel(*refs):
    _lr_body(*refs)


def kernel(user_id, item_id, user_hs, item_hs, beta_u, bias_u,
           beta_i, bias_i, user_weight, item_weight):
    out = _lr_kernel(user_id, item_id, user_hs.reshape(-1), item_hs.reshape(-1),
                     beta_u.reshape(-1), bias_u.reshape(-1),
                     beta_i.reshape(-1), bias_i.reshape(-1),
                     user_weight.reshape(-1), item_weight.reshape(-1))
    return out.reshape(BATCH, 1)
